# Initial kernel scaffold; baseline (speedup 1.0000x reference)
import jax
import jax.numpy as jnp
from jax.experimental import pallas as pl


def kernel(x, w_g, c_fc, c_proj):
    raise NotImplementedError("write your pallas kernel here")


# speedup vs baseline: 2.0576x; 2.0576x over previous
import functools
import math

import jax
import jax.numpy as jnp
from jax import lax
from jax.experimental import pallas as pl
from jax.experimental.pallas import tpu as pltpu
from jax.experimental.pallas import tpu_sc as plsc

N_EXP = 8
TOP_K = 2
N_EMBD = 1024
N_TOK = 2048
CAPACITY = 640
SLOTS = N_EXP * CAPACITY

NC, NS, L = 2, 16, 16
NW = NC * NS
RPT = SLOTS // NW
DCH = 80
TPT = N_TOK // NW
CCH = 32

_BLK = 128
_NBLK = N_TOK // _BLK


def _router_body(x_ref, wg_ref, idxd0_ref, idxd1_ref, idxc0_ref, idxc1_ref,
                 w0_ref, w1_ref):
    x = x_ref[...]
    wg = wg_ref[...]
    logits = lax.dot_general(x, wg, (((1,), (1,)), ((), ())),
                             preferred_element_type=jnp.float32)
    iota_e = lax.broadcasted_iota(jnp.int32, (N_TOK, N_EXP), 1)

    m0 = jnp.max(logits, axis=1, keepdims=True)
    e0 = jnp.min(jnp.where(logits == m0, iota_e, N_EXP), axis=1, keepdims=True)
    oh0 = iota_e == e0
    masked = jnp.where(oh0, -jnp.inf, logits)
    m1 = jnp.max(masked, axis=1, keepdims=True)
    e1 = jnp.min(jnp.where(masked == m1, iota_e, N_EXP), axis=1, keepdims=True)
    oh1 = iota_e == e1

    w0 = 1.0 / (1.0 + jnp.exp(m1 - m0))
    w1 = 1.0 - w0

    bi = lax.broadcasted_iota(jnp.int32, (_BLK, _BLK), 0)
    bj = lax.broadcasted_iota(jnp.int32, (_BLK, _BLK), 1)
    tril = (bi >= bj).astype(jnp.float32)

    def ranks(oh, offset):
        ohf = oh.astype(jnp.float32)
        carry = jnp.zeros((1, N_EXP), jnp.float32)
        blocks = []
        for g in range(_NBLK):
            blk = ohf[g * _BLK:(g + 1) * _BLK, :]
            c = jnp.dot(tril, blk, preferred_element_type=jnp.float32) + carry
            r = jnp.sum(blk * (c + offset), axis=1, keepdims=True)
            blocks.append(r)
            carry = c[_BLK - 1:_BLK, :]
        return jnp.concatenate(blocks, axis=0) - 1.0, carry

    r0f, tot0 = ranks(oh0, jnp.zeros((1, N_EXP), jnp.float32))
    r1f, _ = ranks(oh1, tot0)
    r0 = r0f.astype(jnp.int32)
    r1 = r1f.astype(jnp.int32)

    def emit(e, r, w, idxd_ref, idxc_ref, w_ref):
        valid = r < CAPACITY
        slot = e * CAPACITY + r
        idxd_ref[...] = jnp.where(valid, slot, SLOTS)
        idxc_ref[...] = jnp.where(valid, slot, 0)
        w_ref[...] = jnp.where(valid, w, 0.0)

    emit(e0, r0, w0, idxd0_ref, idxc0_ref, w0_ref)
    emit(e1, r1, w1, idxd1_ref, idxc1_ref, w1_ref)


def _router(xf, w_g):
    i32 = jax.ShapeDtypeStruct((N_TOK, 1), jnp.int32)
    f32 = jax.ShapeDtypeStruct((N_TOK, 1), jnp.float32)
    return pl.pallas_call(
        _router_body,
        out_shape=(i32, i32, i32, i32, f32, f32),
    )(xf, w_g)


def _gelu(h):
    return 0.5 * h * (1.0 + lax.erf(h * (1.0 / math.sqrt(2.0))))


def _mlp_body(x_ref, fc_ref, proj_ref, out_ref):
    f = pl.program_id(1)
    h = jnp.dot(x_ref[0], fc_ref[0], preferred_element_type=jnp.float32)
    h = _gelu(h)
    contrib = jnp.dot(h, proj_ref[0], preferred_element_type=jnp.float32)

    @pl.when(f == 0)
    def _():
        out_ref[0] = contrib

    @pl.when(f != 0)
    def _():
        out_ref[0] = out_ref[0] + contrib


def _mlp(xbuf, c_fc, c_proj):
    fblk = 1024
    nf = 4 * N_EMBD // fblk
    return pl.pallas_call(
        _mlp_body,
        grid=(N_EXP, nf),
        in_specs=[
            pl.BlockSpec((1, CAPACITY, N_EMBD), lambda e, f: (e, 0, 0)),
            pl.BlockSpec((1, N_EMBD, fblk), lambda e, f: (e, 0, f)),
            pl.BlockSpec((1, fblk, N_EMBD), lambda e, f: (e, f, 0)),
        ],
        out_specs=pl.BlockSpec((1, CAPACITY, N_EMBD), lambda e, f: (e, 0, 0)),
        out_shape=jax.ShapeDtypeStruct((N_EXP, CAPACITY, N_EMBD), jnp.float32),
        compiler_params=pltpu.CompilerParams(
            dimension_semantics=("arbitrary", "arbitrary")),
    )(xbuf, c_fc, c_proj)


def _dispatch_body(x_hbm, idx0_hbm, idx1_hbm, xbuf_hbm,
                   i0_v, i1_v, src_v, rows_v, sem):
    wid = lax.axis_index("s") * NC + lax.axis_index("c")
    lo = wid * RPT
    pltpu.sync_copy(idx0_hbm, i0_v)
    pltpu.sync_copy(idx1_hbm, i1_v)

    def init(i, c):
        src_v[pl.ds(i * L, L)] = jnp.zeros((L,), jnp.int32)
        return c

    lax.fori_loop(0, RPT // L, init, 0)
    tok_iota = lax.iota(jnp.int32, L)

    def scan(c, carry):
        base = c * L
        tok = tok_iota + base
        iv0 = i0_v[pl.ds(base, L)]
        plsc.store_scatter(src_v, [iv0 - lo], tok,
                           mask=(iv0 >= lo) & (iv0 < lo + RPT))
        iv1 = i1_v[pl.ds(base, L)]
        plsc.store_scatter(src_v, [iv1 - lo], tok,
                           mask=(iv1 >= lo) & (iv1 < lo + RPT))
        return carry

    lax.fori_loop(0, N_TOK // L, scan, 0)
    for ch in range(RPT // DCH):
        pltpu.async_copy(x_hbm.at[src_v.at[pl.ds(ch * DCH, DCH)]],
                         rows_v, sem).wait()
        pltpu.sync_copy(rows_v, xbuf_hbm.at[pl.ds(lo + ch * DCH, DCH)])


def _dispatch(xf, idx0, idx1):
    return pl.kernel(
        _dispatch_body,
        out_type=jax.ShapeDtypeStruct((SLOTS, N_EMBD), jnp.float32),
        mesh=plsc.VectorSubcoreMesh(core_axis_name="c", subcore_axis_name="s",
                                    num_cores=NC, num_subcores=NS),
        scratch_types=[
            pltpu.VMEM((N_TOK,), jnp.int32),
            pltpu.VMEM((N_TOK,), jnp.int32),
            pltpu.VMEM((RPT,), jnp.int32),
            pltpu.VMEM((DCH, N_EMBD), jnp.float32),
            pltpu.SemaphoreType.DMA,
        ],
        compiler_params=pltpu.CompilerParams(needs_layout_passes=False),
    )(xf, idx0, idx1)


def _combine_body(eo_hbm, idx0_hbm, idx1_hbm, w0_hbm, w1_hbm, out_hbm,
                  i0_v, i1_v, w0_v, w1_v, g0_v, g1_v, ob_v, sem):
    wid = lax.axis_index("s") * NC + lax.axis_index("c")
    t0 = wid * TPT
    pltpu.sync_copy(idx0_hbm.at[pl.ds(t0, TPT)], i0_v)
    pltpu.sync_copy(idx1_hbm.at[pl.ds(t0, TPT)], i1_v)
    pltpu.sync_copy(w0_hbm.at[pl.ds(t0, TPT)], w0_v)
    pltpu.sync_copy(w1_hbm.at[pl.ds(t0, TPT)], w1_v)
    for ch in range(TPT // CCH):
        pltpu.async_copy(eo_hbm.at[i0_v.at[pl.ds(ch * CCH, CCH)]],
                         g0_v, sem).wait()
        pltpu.async_copy(eo_hbm.at[i1_v.at[pl.ds(ch * CCH, CCH)]],
                         g1_v, sem).wait()

        def tbody(t, carry):
            tsplat = jnp.full((L,), ch * CCH + t, jnp.int32)
            w0s = plsc.load_gather(w0_v, [tsplat])
            w1s = plsc.load_gather(w1_v, [tsplat])

            def fbody(fb, c2):
                a = g0_v[t, pl.ds(fb * L, L)]
                b = g1_v[t, pl.ds(fb * L, L)]
                ob_v[t, pl.ds(fb * L, L)] = a * w0s + b * w1s
                return c2

            lax.fori_loop(0, N_EMBD // L, fbody, 0)
            return carry

        lax.fori_loop(0, CCH, tbody, 0)
        pltpu.sync_copy(ob_v, out_hbm.at[pl.ds(t0 + ch * CCH, CCH)])


def _combine(eo, idx0, idx1, w0, w1):
    return pl.kernel(
        _combine_body,
        out_type=jax.ShapeDtypeStruct((N_TOK, N_EMBD), jnp.float32),
        mesh=plsc.VectorSubcoreMesh(core_axis_name="c", subcore_axis_name="s",
                                    num_cores=NC, num_subcores=NS),
        scratch_types=[
            pltpu.VMEM((TPT,), jnp.int32),
            pltpu.VMEM((TPT,), jnp.int32),
            pltpu.VMEM((TPT,), jnp.float32),
            pltpu.VMEM((TPT,), jnp.float32),
            pltpu.VMEM((CCH, N_EMBD), jnp.float32),
            pltpu.VMEM((CCH, N_EMBD), jnp.float32),
            pltpu.VMEM((CCH, N_EMBD), jnp.float32),
            pltpu.SemaphoreType.DMA,
        ],
        compiler_params=pltpu.CompilerParams(needs_layout_passes=False),
    )(eo, idx0, idx1, w0, w1)


def kernel(x, w_g, c_fc, c_proj):
    Bb, Tt, C = x.shape
    xf = x.reshape(N_TOK, N_EMBD)
    idxd0, idxd1, idxc0, idxc1, w0, w1 = _router(xf, w_g)
    xbuf = _dispatch(xf, idxd0.reshape(-1), idxd1.reshape(-1))
    eout = _mlp(xbuf.reshape(N_EXP, CAPACITY, N_EMBD), c_fc, c_proj)
    out = _combine(eout.reshape(SLOTS, N_EMBD),
                   idxc0.reshape(-1), idxc1.reshape(-1),
                   w0.reshape(-1), w1.reshape(-1))
    return out.reshape(Bb, Tt, C)
